# baseline (speedup 1.0000x reference)
import functools

import jax
import jax.numpy as jnp
from jax import lax
from jax.experimental import pallas as pl
from jax.experimental.pallas import tpu as pltpu
from jax.experimental.pallas import tpu_sc as plsc

D = 64
EPS = 1e-5
NC, NS, L = 2, 16, 16
NW = NC * NS
SUBCH = 104
NSUB = 4
CH = SUBCH * NSUB
NBUF = 2


def _rsqrt(v):
    i = plsc.bitcast(v, jnp.int32)
    i = jnp.int32(0x5F3759DF) - lax.shift_right_logical(i, 1)
    y = plsc.bitcast(i, jnp.float32)
    for _ in range(3):
        y = y * (1.5 - 0.5 * v * y * y)
    return y


@functools.lru_cache(maxsize=None)
def _make_kernel(batch, fields):
    n_rows = batch * fields
    rows_per_w = n_rows // NW
    batch_per_w = batch // NW
    n_chunks = rows_per_w // CH
    fd = fields * D
    mesh = plsc.VectorSubcoreMesh(core_axis_name="c", subcore_axis_name="s")

    @functools.partial(
        pl.kernel,
        mesh=mesh,
        compiler_params=pltpu.CompilerParams(
            needs_layout_passes=False, use_tc_tiling_on_sc=False
        ),
        out_type=jax.ShapeDtypeStruct((batch, fields, D), jnp.float32),
        scratch_types=[
            pltpu.VMEM((rows_per_w,), jnp.int32),
            pltpu.VMEM((CH, D), jnp.float32),
            pltpu.VMEM((CH, D), jnp.float32),
            pltpu.VMEM((CH // fields, fields, D), jnp.float32),
            pltpu.VMEM((CH // fields, fields, D), jnp.float32),
            pltpu.VMEM((D,), jnp.float32),
            pltpu.VMEM((D,), jnp.float32),
            pltpu.SemaphoreType.DMA,
            pltpu.SemaphoreType.DMA,
            pltpu.SemaphoreType.DMA,
            pltpu.SemaphoreType.DMA,
        ],
    )
    def run(x_hbm, table_hbm, gamma_hbm, beta_hbm, out_hbm,
            idx_v, in0, in1, ou0, ou1, gam_v, bet_v, gs0, gs1, os0, os1):
        wid = lax.axis_index("s") * NC + lax.axis_index("c")
        base = wid * rows_per_w
        obase = wid * batch_per_w
        pltpu.sync_copy(x_hbm.at[pl.ds(base, rows_per_w)], idx_v)
        pltpu.sync_copy(gamma_hbm, gam_v)
        pltpu.sync_copy(beta_hbm, bet_v)

        ins = (in0, in1)
        outs = (ou0, ou1)
        gsems = (gs0, gs1)
        osems = (os0, os1)
        iota = lax.iota(jnp.int32, L)
        bpc = CH // fields

        def gather_descs(k, b):
            return [
                pltpu.make_async_copy(
                    table_hbm.at[idx_v.at[pl.ds(k * CH + j * SUBCH, SUBCH)]],
                    ins[b].at[pl.ds(j * SUBCH, SUBCH)],
                    gsems[b],
                )
                for j in range(NSUB)
            ]

        def out_desc(k, b):
            return pltpu.make_async_copy(
                outs[b], out_hbm.at[pl.ds(obase + k * bpc, bpc)], osems[b]
            )

        def compute(in_ref, out_ref):
            def pair_body(p, carry):
                rA = p * (2 * L) + iota
                rB = rA + L
                oiA = lax.shift_right_logical(rA * 20165, 19)
                oiB = lax.shift_right_logical(rB * 20165, 19)
                ofA = rA - oiA * fields
                ofB = rB - oiB * fields

                sA = [jnp.zeros((L,), jnp.float32) for _ in range(4)]
                qA = [jnp.zeros((L,), jnp.float32) for _ in range(4)]
                sB = [jnp.zeros((L,), jnp.float32) for _ in range(4)]
                qB = [jnp.zeros((L,), jnp.float32) for _ in range(4)]
                cc = [iota + u for u in range(4)]
                for c in range(D):
                    u = c % 4
                    if c >= 4:
                        cc[u] = (cc[u] + 4) & (D - 1)
                    xA = plsc.load_gather(in_ref, [rA, cc[u]])
                    xB = plsc.load_gather(in_ref, [rB, cc[u]])
                    xA = jnp.maximum(xA, 0.0)
                    xB = jnp.maximum(xB, 0.0)
                    sA[u] += xA
                    qA[u] += xA * xA
                    sB[u] += xB
                    qB[u] += xB * xB

                def finish(s, q):
                    sm = (s[0] + s[1]) + (s[2] + s[3])
                    sq = (q[0] + q[1]) + (q[2] + q[3])
                    mean = sm * (1.0 / D)
                    var = sq * (1.0 / D) - mean * mean
                    a = _rsqrt(var + EPS)
                    return a, -mean * a

                aA, nbA = finish(sA, qA)
                aB, nbB = finish(sB, qB)

                cc = [iota + u for u in range(4)]
                for cb in range(0, D, 4):
                    if cb >= 4:
                        cc = [(c + 4) & (D - 1) for c in cc]
                    gs = [plsc.load_gather(gam_v, [c]) for c in cc]
                    bs = [plsc.load_gather(bet_v, [c]) for c in cc]
                    xA = [plsc.load_gather(in_ref, [rA, c]) for c in cc]
                    xB = [plsc.load_gather(in_ref, [rB, c]) for c in cc]
                    yA = [
                        (jnp.maximum(x, 0.0) * aA + nbA) * g + b
                        for x, g, b in zip(xA, gs, bs)
                    ]
                    yB = [
                        (jnp.maximum(x, 0.0) * aB + nbB) * g + b
                        for x, g, b in zip(xB, gs, bs)
                    ]
                    for u in range(4):
                        plsc.store_scatter(out_ref, [oiA, ofA, cc[u]], yA[u])
                        plsc.store_scatter(out_ref, [oiB, ofB, cc[u]], yB[u])
                return carry

            lax.fori_loop(0, CH // (2 * L), pair_body, 0)

        for d in gather_descs(0, 0):
            d.start()
        for d in gather_descs(1, 1):
            d.start()

        def round_body(g, carry):
            for b in range(NBUF):
                k = g * NBUF + b
                for d in gather_descs(k, b):
                    d.wait()

                @pl.when(k >= NBUF)
                def _():
                    out_desc(k - NBUF, b).wait()

                compute(ins[b], outs[b])
                out_desc(k, b).start()

                @pl.when(k + NBUF < n_chunks)
                def _():
                    for d in gather_descs(k + NBUF, b):
                        d.start()

            return carry

        lax.fori_loop(0, n_chunks // NBUF, round_body, 0)
        out_desc(n_chunks - 2, 0).wait()
        out_desc(n_chunks - 1, 1).wait()

    return run


def kernel(X, table, gamma, beta):
    B, F = X.shape
    idx = X.reshape(B * F).astype(jnp.int32)
    return _make_kernel(B, F)(idx, table, gamma, beta)


# speedup vs baseline: 1.0453x; 1.0453x over previous
import functools

import jax
import jax.numpy as jnp
from jax import lax
from jax.experimental import pallas as pl
from jax.experimental.pallas import tpu as pltpu
from jax.experimental.pallas import tpu_sc as plsc

D = 64
EPS = 1e-5
NC, NS, L = 2, 16, 16
NW = NC * NS
SUBCH = 104
NSUB = 4
CH = SUBCH * NSUB
NBUF = 2


def _rsqrt(v):
    i = plsc.bitcast(v, jnp.int32)
    i = jnp.int32(0x5F3759DF) - lax.shift_right_logical(i, 1)
    y = plsc.bitcast(i, jnp.float32)
    for _ in range(3):
        y = y * (1.5 - 0.5 * v * y * y)
    return y


@functools.lru_cache(maxsize=None)
def _make_kernel(batch, fields):
    n_rows = batch * fields
    rows_per_w = n_rows // NW
    batch_per_w = batch // NW
    n_chunks = rows_per_w // CH
    fd = fields * D
    mesh = plsc.VectorSubcoreMesh(core_axis_name="c", subcore_axis_name="s")

    @functools.partial(
        pl.kernel,
        mesh=mesh,
        compiler_params=pltpu.CompilerParams(
            needs_layout_passes=False, use_tc_tiling_on_sc=False
        ),
        out_type=jax.ShapeDtypeStruct((batch, fd), jnp.float32),
        scratch_types=[
            pltpu.VMEM((rows_per_w,), jnp.int32),
            pltpu.VMEM((CH, D), jnp.float32),
            pltpu.VMEM((CH, D), jnp.float32),
            pltpu.VMEM((CH // fields, fd), jnp.float32),
            pltpu.VMEM((CH // fields, fd), jnp.float32),
            pltpu.VMEM((D,), jnp.float32),
            pltpu.VMEM((D,), jnp.float32),
            pltpu.SemaphoreType.DMA,
            pltpu.SemaphoreType.DMA,
            pltpu.SemaphoreType.DMA,
            pltpu.SemaphoreType.DMA,
        ],
    )
    def run(x_hbm, table_hbm, gamma_hbm, beta_hbm, out_hbm,
            idx_v, in0, in1, ou0, ou1, gam_v, bet_v, gs0, gs1, os0, os1):
        wid = lax.axis_index("s") * NC + lax.axis_index("c")
        base = wid * rows_per_w
        obase = wid * batch_per_w
        pltpu.sync_copy(x_hbm.at[pl.ds(base, rows_per_w)], idx_v)
        pltpu.sync_copy(gamma_hbm, gam_v)
        pltpu.sync_copy(beta_hbm, bet_v)

        ins = (in0, in1)
        outs = (ou0, ou1)
        gsems = (gs0, gs1)
        osems = (os0, os1)
        iota = lax.iota(jnp.int32, L)
        bpc = CH // fields

        def gather_descs(k, b):
            return [
                pltpu.make_async_copy(
                    table_hbm.at[idx_v.at[pl.ds(k * CH + j * SUBCH, SUBCH)]],
                    ins[b].at[pl.ds(j * SUBCH, SUBCH)],
                    gsems[b],
                )
                for j in range(NSUB)
            ]

        def out_desc(k, b):
            return pltpu.make_async_copy(
                outs[b], out_hbm.at[pl.ds(obase + k * bpc, bpc)], osems[b]
            )

        def compute(in_ref, out_ref):
            def pair_body(p, carry):
                rA = p * (2 * L) + iota
                rB = rA + L
                oiA = lax.shift_right_logical(rA * 20165, 19)
                oiB = lax.shift_right_logical(rB * 20165, 19)
                obA = lax.shift_left(rA - oiA * fields, 6)
                obB = lax.shift_left(rB - oiB * fields, 6)

                sA = [jnp.zeros((L,), jnp.float32) for _ in range(4)]
                qA = [jnp.zeros((L,), jnp.float32) for _ in range(4)]
                sB = [jnp.zeros((L,), jnp.float32) for _ in range(4)]
                qB = [jnp.zeros((L,), jnp.float32) for _ in range(4)]
                cc = [iota + u for u in range(4)]
                for c in range(D):
                    u = c % 4
                    if c >= 4:
                        cc[u] = (cc[u] + 4) & (D - 1)
                    xA = plsc.load_gather(in_ref, [rA, cc[u]])
                    xB = plsc.load_gather(in_ref, [rB, cc[u]])
                    xA = jnp.maximum(xA, 0.0)
                    xB = jnp.maximum(xB, 0.0)
                    sA[u] += xA
                    qA[u] += xA * xA
                    sB[u] += xB
                    qB[u] += xB * xB

                def finish(s, q):
                    sm = (s[0] + s[1]) + (s[2] + s[3])
                    sq = (q[0] + q[1]) + (q[2] + q[3])
                    mean = sm * (1.0 / D)
                    var = sq * (1.0 / D) - mean * mean
                    a = _rsqrt(var + EPS)
                    return a, -mean * a

                aA, nbA = finish(sA, qA)
                aB, nbB = finish(sB, qB)

                cc = [iota + u for u in range(4)]
                for cb in range(0, D, 4):
                    if cb >= 4:
                        cc = [(c + 4) & (D - 1) for c in cc]
                    gs = [plsc.load_gather(gam_v, [c]) for c in cc]
                    bs = [plsc.load_gather(bet_v, [c]) for c in cc]
                    xA = [plsc.load_gather(in_ref, [rA, c]) for c in cc]
                    xB = [plsc.load_gather(in_ref, [rB, c]) for c in cc]
                    yA = [
                        (jnp.maximum(x, 0.0) * aA + nbA) * g + b
                        for x, g, b in zip(xA, gs, bs)
                    ]
                    yB = [
                        (jnp.maximum(x, 0.0) * aB + nbB) * g + b
                        for x, g, b in zip(xB, gs, bs)
                    ]
                    for u in range(4):
                        plsc.store_scatter(out_ref, [oiA, obA + cc[u]], yA[u])
                        plsc.store_scatter(out_ref, [oiB, obB + cc[u]], yB[u])
                return carry

            lax.fori_loop(0, CH // (2 * L), pair_body, 0)

        for d in gather_descs(0, 0):
            d.start()
        for d in gather_descs(1, 1):
            d.start()

        def round_body(g, carry):
            for b in range(NBUF):
                k = g * NBUF + b
                for d in gather_descs(k, b):
                    d.wait()

                @pl.when(k >= NBUF)
                def _():
                    out_desc(k - NBUF, b).wait()

                compute(ins[b], outs[b])
                out_desc(k, b).start()

                @pl.when(k + NBUF < n_chunks)
                def _():
                    for d in gather_descs(k + NBUF, b):
                        d.start()

            return carry

        lax.fori_loop(0, n_chunks // NBUF, round_body, 0)
        out_desc(n_chunks - 2, 0).wait()
        out_desc(n_chunks - 1, 1).wait()

    return run


def kernel(X, table, gamma, beta):
    B, F = X.shape
    idx = X.reshape(B * F).astype(jnp.int32)
    out = _make_kernel(B, F)(idx, table, gamma, beta)
    return out.reshape(B, F, D)


# speedup vs baseline: 1.0693x; 1.0230x over previous
import functools

import jax
import jax.numpy as jnp
from jax import lax
from jax.experimental import pallas as pl
from jax.experimental.pallas import tpu as pltpu
from jax.experimental.pallas import tpu_sc as plsc

D = 64
EPS = 1e-5
NC, NS, L = 2, 16, 16
NW = NC * NS
SUBCH = 104
NSUB = 4
CH = SUBCH * NSUB
NBUF = 2


def _rsqrt(v):
    i = plsc.bitcast(v, jnp.int32)
    i = jnp.int32(0x5F3759DF) - lax.shift_right_logical(i, 1)
    y = plsc.bitcast(i, jnp.float32)
    for _ in range(3):
        y = y * (1.5 - 0.5 * v * y * y)
    return y


@functools.lru_cache(maxsize=None)
def _make_kernel(batch, fields):
    n_rows = batch * fields
    rows_per_w = n_rows // NW
    batch_per_w = batch // NW
    n_chunks = rows_per_w // CH
    fd = fields * D
    mesh = plsc.VectorSubcoreMesh(core_axis_name="c", subcore_axis_name="s")

    @functools.partial(
        pl.kernel,
        mesh=mesh,
        compiler_params=pltpu.CompilerParams(
            needs_layout_passes=False, use_tc_tiling_on_sc=False
        ),
        out_type=jax.ShapeDtypeStruct((batch, fd), jnp.float32),
        scratch_types=[
            pltpu.VMEM((rows_per_w,), jnp.int32),
            pltpu.VMEM((CH, D), jnp.float32),
            pltpu.VMEM((CH, D), jnp.float32),
            pltpu.VMEM((CH // fields, fd), jnp.float32),
            pltpu.VMEM((CH // fields, fd), jnp.float32),
            pltpu.VMEM((D,), jnp.float32),
            pltpu.VMEM((D,), jnp.float32),
            pltpu.SemaphoreType.DMA,
            pltpu.SemaphoreType.DMA,
            pltpu.SemaphoreType.DMA,
            pltpu.SemaphoreType.DMA,
        ],
    )
    def run(x_hbm, table_hbm, gamma_hbm, beta_hbm, out_hbm,
            idx_v, in0, in1, ou0, ou1, gam_v, bet_v, gs0, gs1, os0, os1):
        wid = lax.axis_index("s") * NC + lax.axis_index("c")
        base = wid * rows_per_w
        obase = wid * batch_per_w
        pltpu.sync_copy(x_hbm.at[pl.ds(base, rows_per_w)], idx_v)
        pltpu.sync_copy(gamma_hbm, gam_v)
        pltpu.sync_copy(beta_hbm, bet_v)

        ins = (in0, in1)
        outs = (ou0, ou1)
        gsems = (gs0, gs1)
        osems = (os0, os1)
        iota = lax.iota(jnp.int32, L)
        bpc = CH // fields

        def gather_descs(k, b):
            return [
                pltpu.make_async_copy(
                    table_hbm.at[idx_v.at[pl.ds(k * CH + j * SUBCH, SUBCH)]],
                    ins[b].at[pl.ds(j * SUBCH, SUBCH)],
                    gsems[b],
                )
                for j in range(NSUB)
            ]

        def out_desc(k, b):
            return pltpu.make_async_copy(
                outs[b], out_hbm.at[pl.ds(obase + k * bpc, bpc)], osems[b]
            )

        def compute(in_ref, out_ref):
            def pair_body(p, carry):
                rA = p * (2 * L) + iota
                rB = rA + L
                oiA = lax.shift_right_logical(rA * 20165, 19)
                oiB = lax.shift_right_logical(rB * 20165, 19)
                obA = lax.shift_left(rA - oiA * fields, 6)
                obB = lax.shift_left(rB - oiB * fields, 6)

                sA = [jnp.zeros((L,), jnp.float32) for _ in range(2)]
                qA = [jnp.zeros((L,), jnp.float32) for _ in range(2)]
                sB = [jnp.zeros((L,), jnp.float32) for _ in range(2)]
                qB = [jnp.zeros((L,), jnp.float32) for _ in range(2)]
                cc = [iota + u for u in range(4)]
                for c in range(D):
                    u = c % 4
                    if c >= 4:
                        cc[u] = (cc[u] + 4) & (D - 1)
                    xA = plsc.load_gather(in_ref, [rA, cc[u]])
                    xB = plsc.load_gather(in_ref, [rB, cc[u]])
                    xA = jnp.maximum(xA, 0.0)
                    xB = jnp.maximum(xB, 0.0)
                    sA[u % 2] += xA
                    qA[u % 2] += xA * xA
                    sB[u % 2] += xB
                    qB[u % 2] += xB * xB

                def finish(s, q):
                    sm = s[0] + s[1]
                    sq = q[0] + q[1]
                    mean = sm * (1.0 / D)
                    var = sq * (1.0 / D) - mean * mean
                    a = _rsqrt(var + EPS)
                    return a, -mean * a

                aA, nbA = finish(sA, qA)
                aB, nbB = finish(sB, qB)

                cc = [iota + u for u in range(4)]
                for cb in range(0, D, 4):
                    if cb >= 4:
                        cc = [(c + 4) & (D - 1) for c in cc]
                    gs = [plsc.load_gather(gam_v, [c]) for c in cc]
                    bs = [plsc.load_gather(bet_v, [c]) for c in cc]
                    xA = [plsc.load_gather(in_ref, [rA, c]) for c in cc]
                    xB = [plsc.load_gather(in_ref, [rB, c]) for c in cc]
                    yA = [
                        (jnp.maximum(x, 0.0) * aA + nbA) * g + b
                        for x, g, b in zip(xA, gs, bs)
                    ]
                    yB = [
                        (jnp.maximum(x, 0.0) * aB + nbB) * g + b
                        for x, g, b in zip(xB, gs, bs)
                    ]
                    for u in range(4):
                        plsc.store_scatter(out_ref, [oiA, obA + cc[u]], yA[u])
                        plsc.store_scatter(out_ref, [oiB, obB + cc[u]], yB[u])
                return carry

            lax.fori_loop(0, CH // (2 * L), pair_body, 0)

        for d in gather_descs(0, 0):
            d.start()
        for d in gather_descs(1, 1):
            d.start()

        def round_body(g, carry):
            for b in range(NBUF):
                k = g * NBUF + b
                for d in gather_descs(k, b):
                    d.wait()

                @pl.when(k >= NBUF)
                def _():
                    out_desc(k - NBUF, b).wait()

                compute(ins[b], outs[b])
                out_desc(k, b).start()

                @pl.when(k + NBUF < n_chunks)
                def _():
                    for d in gather_descs(k + NBUF, b):
                        d.start()

            return carry

        lax.fori_loop(0, n_chunks // NBUF, round_body, 0)
        out_desc(n_chunks - 2, 0).wait()
        out_desc(n_chunks - 1, 1).wait()

    return run


def kernel(X, table, gamma, beta):
    B, F = X.shape
    idx = X.reshape(B * F).astype(jnp.int32)
    out = _make_kernel(B, F)(idx, table, gamma, beta)
    return out.reshape(B, F, D)
